# baseline (speedup 1.0000x reference)
import functools

import jax
import jax.numpy as jnp
from jax import lax
from jax.experimental import pallas as pl
from jax.experimental.pallas import tpu as pltpu
from jax.experimental.pallas import tpu_sc as plsc

_N = 10000
_E = 320000
_K = 128
_NC, _NS = 2, 16
_NW = _NC * _NS
_CHW = 80
_CH = _CHW * _NW
_EPAD = _CH * _K
_P = 32
_NPAD = 10112
_ROWS_T = _NPAD // _NS
_BM = 1000


def _sc_mesh():
    return plsc.VectorSubcoreMesh(core_axis_name="c", subcore_axis_name="s",
                                  num_cores=_NC, num_subcores=_NS)


def _deg_parts(dstp, ones_blk, zeros_blk):
    @functools.partial(
        pl.kernel,
        out_type=jax.ShapeDtypeStruct((_NC, _NPAD, 128), jnp.float32),
        mesh=_sc_mesh(),
        scratch_types=[
            pltpu.VMEM((_CHW, _K), jnp.int32),
            pltpu.VMEM((_K, 128), jnp.float32),
            pltpu.VMEM_SHARED((_NPAD, 128), jnp.float32),
        ],
    )
    def k(dstp_hbm, ones_hbm, zeros_hbm, out_hbm, dst_v, ones_v, acc):
        c = lax.axis_index("c")
        s = lax.axis_index("s")
        w = s * _NC + c
        pltpu.sync_copy(zeros_hbm, acc.at[pl.ds(s * _ROWS_T, _ROWS_T)])
        pltpu.sync_copy(dstp_hbm.at[pl.ds(w * _CHW, _CHW)], dst_v)
        pltpu.sync_copy(ones_hbm, ones_v)
        plsc.subcore_barrier()

        @pl.loop(0, _CHW)
        def _(j):
            pltpu.sync_copy(ones_v, acc.at[dst_v.at[j]], add=True)

        plsc.subcore_barrier()
        pltpu.sync_copy(acc.at[pl.ds(s * _ROWS_T, _ROWS_T)],
                        out_hbm.at[c, pl.ds(s * _ROWS_T, _ROWS_T)])

    return k(dstp, ones_blk, zeros_blk)


def _agg_parts(g, srcp, dstp, zeros_blk, f, split=(160, 0)):
    c0, c1 = split

    @functools.partial(
        pl.kernel,
        out_type=jax.ShapeDtypeStruct((_NC, _NPAD, f), jnp.float32),
        mesh=_sc_mesh(),
        scratch_types=[
            pltpu.VMEM((_P, _K), jnp.int32),
            pltpu.VMEM((_P, _K), jnp.int32),
            pltpu.VMEM((_K, f), jnp.float32),
            pltpu.VMEM((_K, f), jnp.float32),
            pltpu.VMEM_SHARED((_NPAD, f), jnp.float32),
            pltpu.SemaphoreType.DMA,
            pltpu.SemaphoreType.DMA,
        ],
    )
    def k(g_hbm, srcp_hbm, dstp_hbm, zeros_hbm, out_hbm,
          src_v, dst_v, buf0, buf1, acc, sem0, sem1):
        c = lax.axis_index("c")
        s = lax.axis_index("s")
        pltpu.sync_copy(zeros_hbm, acc.at[pl.ds(s * _ROWS_T, _ROWS_T)])
        plsc.subcore_barrier()

        def stage(b2):
            pltpu.sync_copy(srcp_hbm.at[pl.ds(b2, _P)], src_v)
            pltpu.sync_copy(dstp_hbm.at[pl.ds(b2, _P)], dst_v)
            pltpu.async_copy(g_hbm.at[src_v.at[0]], buf0, sem0)

            @pl.loop(0, _P, step=2)
            def _(j):
                pltpu.make_async_copy(
                    g_hbm.at[src_v.at[j + 1]], buf1, sem1).start()
                pltpu.make_async_copy(g_hbm.at[src_v.at[j]], buf0, sem0).wait()
                pltpu.sync_copy(buf0, acc.at[dst_v.at[j]], add=True)

                @pl.when(j + 2 < _P)
                def _():
                    pltpu.make_async_copy(
                        g_hbm.at[src_v.at[j + 2]], buf0, sem0).start()

                pltpu.make_async_copy(
                    g_hbm.at[src_v.at[j + 1]], buf1, sem1).wait()
                pltpu.sync_copy(buf1, acc.at[dst_v.at[j + 1]], add=True)

        @pl.when(c == 0)
        def _():
            for st in range(c0 // _P):
                stage(s * c0 + st * _P)

        @pl.when(c == 1)
        def _():
            for st in range(c1 // _P):
                stage(_NS * c0 + s * c1 + st * _P)

        plsc.subcore_barrier()
        pltpu.sync_copy(acc.at[pl.ds(s * _ROWS_T, _ROWS_T)],
                        out_hbm.at[c, pl.ds(s * _ROWS_T, _ROWS_T)])

    return k(g, srcp, dstp, zeros_blk)


def _dis_body(degp_ref, dis_ref):
    deg = degp_ref[0, :, 0:1] + degp_ref[1, :, 0:1] + 1.0
    dis_ref[...] = lax.rsqrt(deg)


def _dis(degp):
    grid = _NPAD // _ROWS_T
    return pl.pallas_call(
        _dis_body,
        grid=(grid,),
        in_specs=[pl.BlockSpec((_NC, _ROWS_T, 128), lambda i: (0, i, 0))],
        out_specs=pl.BlockSpec((_ROWS_T, 1), lambda i: (i, 0)),
        out_shape=jax.ShapeDtypeStruct((_NPAD, 1), jnp.float32),
    )(degp)


def _mm1_body(x_ref, w_ref, dis_ref, g_ref):
    g_ref[...] = jnp.dot(x_ref[...], w_ref[...],
                         preferred_element_type=jnp.float32) * dis_ref[...]


def _mm1(x, w1, dis):
    grid = _N // _BM
    return pl.pallas_call(
        _mm1_body,
        grid=(grid,),
        in_specs=[
            pl.BlockSpec((_BM, 128), lambda i: (i, 0)),
            pl.BlockSpec((128, 128), lambda i: (0, 0)),
            pl.BlockSpec((_BM, 1), lambda i: (i, 0)),
        ],
        out_specs=pl.BlockSpec((_BM, 128), lambda i: (i, 0)),
        out_shape=jax.ShapeDtypeStruct((_N, 128), jnp.float32),
    )(x, w1, dis)


def _combine_mm_body(parts_ref, g_ref, dis_ref, b_ref, w_ref, o_ref):
    z = dis_ref[...] * (parts_ref[0] + parts_ref[1] + g_ref[...]) + b_ref[...]
    a = jnp.maximum(z, 0.0)
    o_ref[...] = jnp.dot(a, w_ref[...],
                         preferred_element_type=jnp.float32) * dis_ref[...]


def _combine_mm(parts, g, dis, b, w):
    grid = _N // _BM
    fin = g.shape[1]
    fout = w.shape[1]
    return pl.pallas_call(
        _combine_mm_body,
        grid=(grid,),
        in_specs=[
            pl.BlockSpec((_NC, _BM, fin), lambda i: (0, i, 0)),
            pl.BlockSpec((_BM, fin), lambda i: (i, 0)),
            pl.BlockSpec((_BM, 1), lambda i: (i, 0)),
            pl.BlockSpec((1, fin), lambda i: (0, 0)),
            pl.BlockSpec((fin, fout), lambda i: (0, 0)),
        ],
        out_specs=pl.BlockSpec((_BM, fout), lambda i: (i, 0)),
        out_shape=jax.ShapeDtypeStruct((_N, fout), jnp.float32),
    )(parts, g, dis, b, w)


def _combine_scale_body(parts_ref, g_ref, dis_ref, b_ref, o_ref):
    z = dis_ref[...] * (parts_ref[0] + parts_ref[1] + g_ref[...]) + b_ref[...]
    o_ref[...] = jnp.maximum(z, 0.0) * dis_ref[...]


def _combine_scale(parts, g, dis, b):
    grid = _N // _BM
    f = g.shape[1]
    return pl.pallas_call(
        _combine_scale_body,
        grid=(grid,),
        in_specs=[
            pl.BlockSpec((_NC, _BM, f), lambda i: (0, i, 0)),
            pl.BlockSpec((_BM, f), lambda i: (i, 0)),
            pl.BlockSpec((_BM, 1), lambda i: (i, 0)),
            pl.BlockSpec((1, f), lambda i: (0, 0)),
        ],
        out_specs=pl.BlockSpec((_BM, f), lambda i: (i, 0)),
        out_shape=jax.ShapeDtypeStruct((_N, f), jnp.float32),
    )(parts, g, dis, b)


def _final_body(parts_ref, q_ref, dis_ref, w_ref, b_ref, o_ref):
    a = parts_ref[0] + parts_ref[1] + q_ref[...]
    z = jnp.dot(a, w_ref[...], preferred_element_type=jnp.float32)
    z = z * dis_ref[...] + b_ref[...]
    m = jnp.max(z, axis=1, keepdims=True)
    lse = jnp.log(jnp.sum(jnp.exp(z - m), axis=1, keepdims=True)) + m
    o_ref[...] = z - lse


def _final(parts, q, dis, w2, b2):
    grid = _N // _BM
    f = q.shape[1]
    cls = w2.shape[1]
    return pl.pallas_call(
        _final_body,
        grid=(grid,),
        in_specs=[
            pl.BlockSpec((_NC, _BM, f), lambda i: (0, i, 0)),
            pl.BlockSpec((_BM, f), lambda i: (i, 0)),
            pl.BlockSpec((_BM, 1), lambda i: (i, 0)),
            pl.BlockSpec((f, cls), lambda i: (0, 0)),
            pl.BlockSpec((1, cls), lambda i: (0, 0)),
        ],
        out_specs=pl.BlockSpec((_BM, cls), lambda i: (i, 0)),
        out_shape=jax.ShapeDtypeStruct((_N, cls), jnp.float32),
    )(parts, q, dis, w2, b2)


def kernel(x, edge_index, W1, b1, Wg0, bg0, Wg1, bg1, W2, b2):
    src, dst = edge_index[0], edge_index[1]
    npad = _EPAD - _E
    srcp = jnp.concatenate([src, jnp.zeros((npad,), jnp.int32)]).reshape(_CH, _K)
    dstp = jnp.concatenate([dst, jnp.full((npad,), _N, jnp.int32)]).reshape(_CH, _K)
    zeros128 = jnp.zeros((_ROWS_T, 128), jnp.float32)
    ones128 = jnp.ones((_K, 128), jnp.float32)

    degp = _deg_parts(dstp, ones128, zeros128)
    dis = _dis(degp)
    g1 = _mm1(x, W1, dis)

    parts = _agg_parts(g1, srcp, dstp, zeros128, 128)
    g2 = _combine_mm(parts, g1, dis, b1.reshape(1, -1), Wg0)

    parts = _agg_parts(g2, srcp, dstp, zeros128, 128)
    g3 = _combine_mm(parts, g2, dis, bg0.reshape(1, -1), Wg1)

    parts = _agg_parts(g3, srcp, dstp, zeros128, 128)
    q = _combine_scale(parts, g3, dis, bg1.reshape(1, -1))

    parts = _agg_parts(q, srcp, dstp, zeros128, 128)
    out = _final(parts, q, dis, W2, b2.reshape(1, -1))
    return out


# speedup vs baseline: 3.5771x; 3.5771x over previous
import functools

import jax
import jax.numpy as jnp
from jax import lax
from jax.experimental import pallas as pl
from jax.experimental.pallas import tpu as pltpu
from jax.experimental.pallas import tpu_sc as plsc

_N = 10000
_E = 320000
_K = 128
_NC, _NS = 2, 16
_NW = _NC * _NS
_CHW = 80
_CH = _CHW * _NW
_EPAD = _CH * _K
_P = 40
_NPAD = 10112
_ROWS_T = _NPAD // _NS
_BM = 1000


def _sc_mesh():
    return plsc.VectorSubcoreMesh(core_axis_name="c", subcore_axis_name="s",
                                  num_cores=_NC, num_subcores=_NS)


def _deg_parts(dstp, ones_blk, zeros_blk):
    @functools.partial(
        pl.kernel,
        out_type=jax.ShapeDtypeStruct((_NC, _NPAD, 128), jnp.float32),
        mesh=_sc_mesh(),
        scratch_types=[
            pltpu.VMEM((_CHW, _K), jnp.int32),
            pltpu.VMEM((_K, 128), jnp.float32),
            pltpu.VMEM_SHARED((_NPAD, 128), jnp.float32),
        ],
    )
    def k(dstp_hbm, ones_hbm, zeros_hbm, out_hbm, dst_v, ones_v, acc):
        c = lax.axis_index("c")
        s = lax.axis_index("s")
        w = s * _NC + c
        pltpu.sync_copy(zeros_hbm, acc.at[pl.ds(s * _ROWS_T, _ROWS_T)])
        pltpu.sync_copy(dstp_hbm.at[pl.ds(w * _CHW, _CHW)], dst_v)
        pltpu.sync_copy(ones_hbm, ones_v)
        plsc.subcore_barrier()

        @pl.loop(0, _CHW)
        def _(j):
            pltpu.sync_copy(ones_v, acc.at[dst_v.at[j]], add=True)

        plsc.subcore_barrier()
        pltpu.sync_copy(acc.at[pl.ds(s * _ROWS_T, _ROWS_T)],
                        out_hbm.at[c, pl.ds(s * _ROWS_T, _ROWS_T)])

    return k(dstp, ones_blk, zeros_blk)


def _agg_parts(g, srcp, dstp, zeros_blk, f, split=(80, 80)):
    c0, c1 = split

    @functools.partial(
        pl.kernel,
        out_type=jax.ShapeDtypeStruct((_NC, _NPAD, f), jnp.float32),
        mesh=_sc_mesh(),
        scratch_types=[
            pltpu.VMEM((_P, _K), jnp.int32),
            pltpu.VMEM((_P, _K), jnp.int32),
            pltpu.VMEM((_K, f), jnp.float32),
            pltpu.VMEM((_K, f), jnp.float32),
            pltpu.VMEM_SHARED((_NPAD, f), jnp.float32),
            pltpu.SemaphoreType.DMA,
            pltpu.SemaphoreType.DMA,
        ],
    )
    def k(g_hbm, srcp_hbm, dstp_hbm, zeros_hbm, out_hbm,
          src_v, dst_v, buf0, buf1, acc, sem0, sem1):
        c = lax.axis_index("c")
        s = lax.axis_index("s")
        pltpu.sync_copy(zeros_hbm, acc.at[pl.ds(s * _ROWS_T, _ROWS_T)])
        plsc.subcore_barrier()

        def stage(b2):
            pltpu.sync_copy(srcp_hbm.at[pl.ds(b2, _P)], src_v)
            pltpu.sync_copy(dstp_hbm.at[pl.ds(b2, _P)], dst_v)
            pltpu.async_copy(g_hbm.at[src_v.at[0]], buf0, sem0)

            @pl.loop(0, _P, step=2)
            def _(j):
                pltpu.make_async_copy(
                    g_hbm.at[src_v.at[j + 1]], buf1, sem1).start()
                pltpu.make_async_copy(g_hbm.at[src_v.at[j]], buf0, sem0).wait()
                pltpu.sync_copy(buf0, acc.at[dst_v.at[j]], add=True)

                @pl.when(j + 2 < _P)
                def _():
                    pltpu.make_async_copy(
                        g_hbm.at[src_v.at[j + 2]], buf0, sem0).start()

                pltpu.make_async_copy(
                    g_hbm.at[src_v.at[j + 1]], buf1, sem1).wait()
                pltpu.sync_copy(buf1, acc.at[dst_v.at[j + 1]], add=True)

        @pl.when(c == 0)
        def _():
            for st in range(c0 // _P):
                stage(s * c0 + st * _P)

        @pl.when(c == 1)
        def _():
            for st in range(c1 // _P):
                stage(_NS * c0 + s * c1 + st * _P)

        plsc.subcore_barrier()
        pltpu.sync_copy(acc.at[pl.ds(s * _ROWS_T, _ROWS_T)],
                        out_hbm.at[c, pl.ds(s * _ROWS_T, _ROWS_T)])

    return k(g, srcp, dstp, zeros_blk)


def _dis_body(degp_ref, dis_ref):
    deg = degp_ref[0, :, 0:1] + degp_ref[1, :, 0:1] + 1.0
    dis_ref[...] = lax.rsqrt(deg)


def _dis(degp):
    grid = _NPAD // _ROWS_T
    return pl.pallas_call(
        _dis_body,
        grid=(grid,),
        in_specs=[pl.BlockSpec((_NC, _ROWS_T, 128), lambda i: (0, i, 0))],
        out_specs=pl.BlockSpec((_ROWS_T, 1), lambda i: (i, 0)),
        out_shape=jax.ShapeDtypeStruct((_NPAD, 1), jnp.float32),
    )(degp)


def _mm1_body(x_ref, w_ref, dis_ref, g_ref):
    g_ref[...] = jnp.dot(x_ref[...], w_ref[...],
                         preferred_element_type=jnp.float32) * dis_ref[...]


def _mm1(x, w1, dis):
    grid = _N // _BM
    return pl.pallas_call(
        _mm1_body,
        grid=(grid,),
        in_specs=[
            pl.BlockSpec((_BM, 128), lambda i: (i, 0)),
            pl.BlockSpec((128, 128), lambda i: (0, 0)),
            pl.BlockSpec((_BM, 1), lambda i: (i, 0)),
        ],
        out_specs=pl.BlockSpec((_BM, 128), lambda i: (i, 0)),
        out_shape=jax.ShapeDtypeStruct((_N, 128), jnp.float32),
    )(x, w1, dis)


def _combine_mm_body(parts_ref, g_ref, dis_ref, b_ref, w_ref, o_ref):
    z = dis_ref[...] * (parts_ref[0] + parts_ref[1] + g_ref[...]) + b_ref[...]
    a = jnp.maximum(z, 0.0)
    o_ref[...] = jnp.dot(a, w_ref[...],
                         preferred_element_type=jnp.float32) * dis_ref[...]


def _combine_mm(parts, g, dis, b, w):
    grid = _N // _BM
    fin = g.shape[1]
    fout = w.shape[1]
    return pl.pallas_call(
        _combine_mm_body,
        grid=(grid,),
        in_specs=[
            pl.BlockSpec((_NC, _BM, fin), lambda i: (0, i, 0)),
            pl.BlockSpec((_BM, fin), lambda i: (i, 0)),
            pl.BlockSpec((_BM, 1), lambda i: (i, 0)),
            pl.BlockSpec((1, fin), lambda i: (0, 0)),
            pl.BlockSpec((fin, fout), lambda i: (0, 0)),
        ],
        out_specs=pl.BlockSpec((_BM, fout), lambda i: (i, 0)),
        out_shape=jax.ShapeDtypeStruct((_N, fout), jnp.float32),
    )(parts, g, dis, b, w)


def _combine_scale_body(parts_ref, g_ref, dis_ref, b_ref, o_ref):
    z = dis_ref[...] * (parts_ref[0] + parts_ref[1] + g_ref[...]) + b_ref[...]
    o_ref[...] = jnp.maximum(z, 0.0) * dis_ref[...]


def _combine_scale(parts, g, dis, b):
    grid = _N // _BM
    f = g.shape[1]
    return pl.pallas_call(
        _combine_scale_body,
        grid=(grid,),
        in_specs=[
            pl.BlockSpec((_NC, _BM, f), lambda i: (0, i, 0)),
            pl.BlockSpec((_BM, f), lambda i: (i, 0)),
            pl.BlockSpec((_BM, 1), lambda i: (i, 0)),
            pl.BlockSpec((1, f), lambda i: (0, 0)),
        ],
        out_specs=pl.BlockSpec((_BM, f), lambda i: (i, 0)),
        out_shape=jax.ShapeDtypeStruct((_N, f), jnp.float32),
    )(parts, g, dis, b)


def _final_body(parts_ref, q_ref, dis_ref, w_ref, b_ref, o_ref):
    a = parts_ref[0] + parts_ref[1] + q_ref[...]
    z = jnp.dot(a, w_ref[...], preferred_element_type=jnp.float32)
    z = z * dis_ref[...] + b_ref[...]
    m = jnp.max(z, axis=1, keepdims=True)
    lse = jnp.log(jnp.sum(jnp.exp(z - m), axis=1, keepdims=True)) + m
    o_ref[...] = z - lse


def _final(parts, q, dis, w2, b2):
    grid = _N // _BM
    f = q.shape[1]
    cls = w2.shape[1]
    return pl.pallas_call(
        _final_body,
        grid=(grid,),
        in_specs=[
            pl.BlockSpec((_NC, _BM, f), lambda i: (0, i, 0)),
            pl.BlockSpec((_BM, f), lambda i: (i, 0)),
            pl.BlockSpec((_BM, 1), lambda i: (i, 0)),
            pl.BlockSpec((f, cls), lambda i: (0, 0)),
            pl.BlockSpec((1, cls), lambda i: (0, 0)),
        ],
        out_specs=pl.BlockSpec((_BM, cls), lambda i: (i, 0)),
        out_shape=jax.ShapeDtypeStruct((_N, cls), jnp.float32),
    )(parts, q, dis, w2, b2)


def kernel(x, edge_index, W1, b1, Wg0, bg0, Wg1, bg1, W2, b2):
    src, dst = edge_index[0], edge_index[1]
    npad = _EPAD - _E
    iota = jnp.arange(npad, dtype=jnp.int32)
    srcp = jnp.concatenate([src, iota % _N]).reshape(_CH, _K)
    dstp = jnp.concatenate([dst, _N + iota % (_NPAD - _N)]).reshape(_CH, _K)
    zeros128 = jnp.zeros((_ROWS_T, 128), jnp.float32)
    ones128 = jnp.ones((_K, 128), jnp.float32)

    degp = _deg_parts(dstp, ones128, zeros128)
    dis = _dis(degp)
    g1 = _mm1(x, W1, dis)

    parts = _agg_parts(g1, srcp, dstp, zeros128, 128)
    g2 = _combine_mm(parts, g1, dis, b1.reshape(1, -1), Wg0)

    parts = _agg_parts(g2, srcp, dstp, zeros128, 128)
    g3 = _combine_mm(parts, g2, dis, bg0.reshape(1, -1), Wg1)

    parts = _agg_parts(g3, srcp, dstp, zeros128, 128)
    q = _combine_scale(parts, g3, dis, bg1.reshape(1, -1))

    parts = _agg_parts(q, srcp, dstp, zeros128, 128)
    out = _final(parts, q, dis, W2, b2.reshape(1, -1))
    return out


# speedup vs baseline: 3.6738x; 1.0270x over previous
import functools

import jax
import jax.numpy as jnp
from jax import lax
from jax.experimental import pallas as pl
from jax.experimental.pallas import tpu as pltpu
from jax.experimental.pallas import tpu_sc as plsc

_N = 10000
_E = 320000
_K = 64
_NC, _NS = 2, 16
_NW = _NC * _NS
_CHW = 160
_CH = _CHW * _NW
_EPAD = _CH * _K
_P = 40
_NPAD = 10112
_ROWS_T = _NPAD // _NS
_BM = 1000


def _sc_mesh():
    return plsc.VectorSubcoreMesh(core_axis_name="c", subcore_axis_name="s",
                                  num_cores=_NC, num_subcores=_NS)


def _deg_parts(dstp, ones_blk, zeros_blk):
    @functools.partial(
        pl.kernel,
        out_type=jax.ShapeDtypeStruct((_NC, _NPAD, 128), jnp.float32),
        mesh=_sc_mesh(),
        scratch_types=[
            pltpu.VMEM((_CHW, _K), jnp.int32),
            pltpu.VMEM((_K, 128), jnp.float32),
            pltpu.VMEM_SHARED((_NPAD, 128), jnp.float32),
        ],
    )
    def k(dstp_hbm, ones_hbm, zeros_hbm, out_hbm, dst_v, ones_v, acc):
        c = lax.axis_index("c")
        s = lax.axis_index("s")
        w = s * _NC + c
        pltpu.sync_copy(zeros_hbm, acc.at[pl.ds(s * _ROWS_T, _ROWS_T)])
        pltpu.sync_copy(dstp_hbm.at[pl.ds(w * _CHW, _CHW)], dst_v)
        pltpu.sync_copy(ones_hbm, ones_v)
        plsc.subcore_barrier()

        @pl.loop(0, _CHW)
        def _(j):
            pltpu.sync_copy(ones_v, acc.at[dst_v.at[j]], add=True)

        plsc.subcore_barrier()
        pltpu.sync_copy(acc.at[pl.ds(s * _ROWS_T, _ROWS_T)],
                        out_hbm.at[c, pl.ds(s * _ROWS_T, _ROWS_T)])

    return k(dstp, ones_blk, zeros_blk)


def _agg_parts(g, srcp, dstp, zeros_blk, f, split=(160, 160)):
    c0, c1 = split

    nb = 4

    @functools.partial(
        pl.kernel,
        out_type=jax.ShapeDtypeStruct((_NC, _NPAD, f), jnp.float32),
        mesh=_sc_mesh(),
        scratch_types=[
            pltpu.VMEM((_P, _K), jnp.int32),
            pltpu.VMEM((_P, _K), jnp.int32),
            [pltpu.VMEM((_K, f), jnp.float32) for _ in range(nb)],
            pltpu.VMEM_SHARED((_NPAD, f), jnp.float32),
            [pltpu.SemaphoreType.DMA for _ in range(nb)],
            [pltpu.SemaphoreType.DMA for _ in range(nb)],
        ],
    )
    def k(g_hbm, srcp_hbm, dstp_hbm, zeros_hbm, out_hbm,
          src_v, dst_v, bufs, acc, gsem, ssem):
        c = lax.axis_index("c")
        s = lax.axis_index("s")
        pltpu.sync_copy(zeros_hbm, acc.at[pl.ds(s * _ROWS_T, _ROWS_T)])
        plsc.subcore_barrier()

        def gather(cc, b):
            return pltpu.make_async_copy(g_hbm.at[src_v.at[cc]], bufs[b],
                                         gsem[b])

        def scatter(cc, b):
            return pltpu.make_async_copy(bufs[b], acc.at[dst_v.at[cc]],
                                         ssem[b])

        def stage(b2):
            pltpu.sync_copy(srcp_hbm.at[pl.ds(b2, _P)], src_v)
            pltpu.sync_copy(dstp_hbm.at[pl.ds(b2, _P)], dst_v)
            for b in range(nb - 1):
                gather(b, b).start()

            @pl.loop(0, _P, step=nb)
            def _(j):
                for b in range(nb):
                    cc = j + b
                    gather(cc, b).wait()
                    scatter(cc, b).start()
                    tb = (b + nb - 1) % nb

                    @pl.when(cc + nb - 1 < _P)
                    def _():
                        if b == 0:
                            @pl.when(j > 0)
                            def _():
                                scatter(cc - 1, tb).wait()
                        else:
                            scatter(cc - 1, tb).wait()
                        gather(cc + nb - 1, tb).start()

            for b in range(nb):
                scatter(_P - nb + b, b).wait()

        @pl.when(c == 0)
        def _():
            for st in range(c0 // _P):
                stage(s * c0 + st * _P)

        @pl.when(c == 1)
        def _():
            for st in range(c1 // _P):
                stage(_NS * c0 + s * c1 + st * _P)

        plsc.subcore_barrier()
        pltpu.sync_copy(acc.at[pl.ds(s * _ROWS_T, _ROWS_T)],
                        out_hbm.at[c, pl.ds(s * _ROWS_T, _ROWS_T)])

    return k(g, srcp, dstp, zeros_blk)


def _dis_body(degp_ref, dis_ref):
    deg = degp_ref[0, :, 0:1] + degp_ref[1, :, 0:1] + 1.0
    dis_ref[...] = lax.rsqrt(deg)


def _dis(degp):
    grid = _NPAD // _ROWS_T
    return pl.pallas_call(
        _dis_body,
        grid=(grid,),
        in_specs=[pl.BlockSpec((_NC, _ROWS_T, 128), lambda i: (0, i, 0))],
        out_specs=pl.BlockSpec((_ROWS_T, 1), lambda i: (i, 0)),
        out_shape=jax.ShapeDtypeStruct((_NPAD, 1), jnp.float32),
    )(degp)


def _mm1_body(x_ref, w_ref, dis_ref, g_ref):
    g_ref[...] = jnp.dot(x_ref[...], w_ref[...],
                         preferred_element_type=jnp.float32) * dis_ref[...]


def _mm1(x, w1, dis):
    grid = _N // _BM
    return pl.pallas_call(
        _mm1_body,
        grid=(grid,),
        in_specs=[
            pl.BlockSpec((_BM, 128), lambda i: (i, 0)),
            pl.BlockSpec((128, 128), lambda i: (0, 0)),
            pl.BlockSpec((_BM, 1), lambda i: (i, 0)),
        ],
        out_specs=pl.BlockSpec((_BM, 128), lambda i: (i, 0)),
        out_shape=jax.ShapeDtypeStruct((_N, 128), jnp.float32),
    )(x, w1, dis)


def _combine_mm_body(parts_ref, g_ref, dis_ref, b_ref, w_ref, o_ref):
    z = dis_ref[...] * (parts_ref[0] + parts_ref[1] + g_ref[...]) + b_ref[...]
    a = jnp.maximum(z, 0.0)
    o_ref[...] = jnp.dot(a, w_ref[...],
                         preferred_element_type=jnp.float32) * dis_ref[...]


def _combine_mm(parts, g, dis, b, w):
    grid = _N // _BM
    fin = g.shape[1]
    fout = w.shape[1]
    return pl.pallas_call(
        _combine_mm_body,
        grid=(grid,),
        in_specs=[
            pl.BlockSpec((_NC, _BM, fin), lambda i: (0, i, 0)),
            pl.BlockSpec((_BM, fin), lambda i: (i, 0)),
            pl.BlockSpec((_BM, 1), lambda i: (i, 0)),
            pl.BlockSpec((1, fin), lambda i: (0, 0)),
            pl.BlockSpec((fin, fout), lambda i: (0, 0)),
        ],
        out_specs=pl.BlockSpec((_BM, fout), lambda i: (i, 0)),
        out_shape=jax.ShapeDtypeStruct((_N, fout), jnp.float32),
    )(parts, g, dis, b, w)


def _combine_scale_body(parts_ref, g_ref, dis_ref, b_ref, o_ref):
    z = dis_ref[...] * (parts_ref[0] + parts_ref[1] + g_ref[...]) + b_ref[...]
    o_ref[...] = jnp.maximum(z, 0.0) * dis_ref[...]


def _combine_scale(parts, g, dis, b):
    grid = _N // _BM
    f = g.shape[1]
    return pl.pallas_call(
        _combine_scale_body,
        grid=(grid,),
        in_specs=[
            pl.BlockSpec((_NC, _BM, f), lambda i: (0, i, 0)),
            pl.BlockSpec((_BM, f), lambda i: (i, 0)),
            pl.BlockSpec((_BM, 1), lambda i: (i, 0)),
            pl.BlockSpec((1, f), lambda i: (0, 0)),
        ],
        out_specs=pl.BlockSpec((_BM, f), lambda i: (i, 0)),
        out_shape=jax.ShapeDtypeStruct((_N, f), jnp.float32),
    )(parts, g, dis, b)


def _final_body(parts_ref, q_ref, dis_ref, w_ref, b_ref, o_ref):
    a = parts_ref[0] + parts_ref[1] + q_ref[...]
    z = jnp.dot(a, w_ref[...], preferred_element_type=jnp.float32)
    z = z * dis_ref[...] + b_ref[...]
    m = jnp.max(z, axis=1, keepdims=True)
    lse = jnp.log(jnp.sum(jnp.exp(z - m), axis=1, keepdims=True)) + m
    o_ref[...] = z - lse


def _final(parts, q, dis, w2, b2):
    grid = _N // _BM
    f = q.shape[1]
    cls = w2.shape[1]
    return pl.pallas_call(
        _final_body,
        grid=(grid,),
        in_specs=[
            pl.BlockSpec((_NC, _BM, f), lambda i: (0, i, 0)),
            pl.BlockSpec((_BM, f), lambda i: (i, 0)),
            pl.BlockSpec((_BM, 1), lambda i: (i, 0)),
            pl.BlockSpec((f, cls), lambda i: (0, 0)),
            pl.BlockSpec((1, cls), lambda i: (0, 0)),
        ],
        out_specs=pl.BlockSpec((_BM, cls), lambda i: (i, 0)),
        out_shape=jax.ShapeDtypeStruct((_N, cls), jnp.float32),
    )(parts, q, dis, w2, b2)


def kernel(x, edge_index, W1, b1, Wg0, bg0, Wg1, bg1, W2, b2):
    src, dst = edge_index[0], edge_index[1]
    npad = _EPAD - _E
    iota = jnp.arange(npad, dtype=jnp.int32)
    srcp = jnp.concatenate([src, iota % _N]).reshape(_CH, _K)
    dstp = jnp.concatenate([dst, _N + iota % (_NPAD - _N)]).reshape(_CH, _K)
    zeros128 = jnp.zeros((_ROWS_T, 128), jnp.float32)
    ones128 = jnp.ones((_K, 128), jnp.float32)

    degp = _deg_parts(dstp, ones128, zeros128)
    dis = _dis(degp)
    g1 = _mm1(x, W1, dis)

    parts = _agg_parts(g1, srcp, dstp, zeros128, 128)
    g2 = _combine_mm(parts, g1, dis, b1.reshape(1, -1), Wg0)

    parts = _agg_parts(g2, srcp, dstp, zeros128, 128)
    g3 = _combine_mm(parts, g2, dis, bg0.reshape(1, -1), Wg1)

    parts = _agg_parts(g3, srcp, dstp, zeros128, 128)
    q = _combine_scale(parts, g3, dis, bg1.reshape(1, -1))

    parts = _agg_parts(q, srcp, dstp, zeros128, 128)
    out = _final(parts, q, dis, W2, b2.reshape(1, -1))
    return out


# speedup vs baseline: 3.9188x; 1.0667x over previous
import functools

import jax
import jax.numpy as jnp
from jax import lax
from jax.experimental import pallas as pl
from jax.experimental.pallas import tpu as pltpu
from jax.experimental.pallas import tpu_sc as plsc

_N = 10000
_E = 320000
_K = 64
_NC, _NS = 2, 16
_NW = _NC * _NS
_CHW = 160
_CH = _CHW * _NW
_EPAD = _CH * _K
_P = 40
_NPAD = 10112
_ROWS_T = _NPAD // _NS
_BM = 1000


def _sc_mesh():
    return plsc.VectorSubcoreMesh(core_axis_name="c", subcore_axis_name="s",
                                  num_cores=_NC, num_subcores=_NS)


def _deg_parts(dstp, ones_blk, zeros_blk):
    @functools.partial(
        pl.kernel,
        out_type=jax.ShapeDtypeStruct((_NC, _NPAD, 128), jnp.float32),
        mesh=_sc_mesh(),
        scratch_types=[
            pltpu.VMEM((_CHW, _K), jnp.int32),
            pltpu.VMEM((_K, 128), jnp.float32),
            pltpu.VMEM_SHARED((_NPAD, 128), jnp.float32),
        ],
    )
    def k(dstp_hbm, ones_hbm, zeros_hbm, out_hbm, dst_v, ones_v, acc):
        c = lax.axis_index("c")
        s = lax.axis_index("s")
        w = s * _NC + c
        pltpu.sync_copy(zeros_hbm, acc.at[pl.ds(s * _ROWS_T, _ROWS_T)])
        pltpu.sync_copy(dstp_hbm.at[pl.ds(w * _CHW, _CHW)], dst_v)
        pltpu.sync_copy(ones_hbm, ones_v)
        plsc.subcore_barrier()

        @pl.loop(0, _CHW)
        def _(j):
            pltpu.sync_copy(ones_v, acc.at[dst_v.at[j]], add=True)

        plsc.subcore_barrier()
        pltpu.sync_copy(acc.at[pl.ds(s * _ROWS_T, _ROWS_T)],
                        out_hbm.at[c, pl.ds(s * _ROWS_T, _ROWS_T)])

    return k(dstp, ones_blk, zeros_blk)


def _agg_parts(g, srcp, dstp, zeros_blk, f, split=(160, 160)):
    c0, c1 = split

    nb = 4

    @functools.partial(
        pl.kernel,
        out_type=jax.ShapeDtypeStruct((_NC, _NPAD, f), jnp.float32),
        mesh=_sc_mesh(),
        scratch_types=[
            pltpu.VMEM((_P, _K), jnp.int32),
            pltpu.VMEM((_P, _K), jnp.int32),
            [pltpu.VMEM((_K, f), jnp.float32) for _ in range(nb)],
            pltpu.VMEM_SHARED((_NPAD, f), jnp.float32),
            [pltpu.SemaphoreType.DMA for _ in range(nb)],
            [pltpu.SemaphoreType.DMA for _ in range(nb)],
        ],
    )
    def k(g_hbm, srcp_hbm, dstp_hbm, zeros_hbm, out_hbm,
          src_v, dst_v, bufs, acc, gsem, ssem):
        c = lax.axis_index("c")
        s = lax.axis_index("s")
        pltpu.sync_copy(zeros_hbm, acc.at[pl.ds(s * _ROWS_T, _ROWS_T)])
        plsc.subcore_barrier()

        def gather(cc, b):
            return pltpu.make_async_copy(g_hbm.at[src_v.at[cc]], bufs[b],
                                         gsem[b])

        def scatter(cc, b):
            return pltpu.make_async_copy(bufs[b], acc.at[dst_v.at[cc]],
                                         ssem[b])

        def stage(b2):
            pltpu.sync_copy(srcp_hbm.at[pl.ds(b2, _P)], src_v)
            pltpu.sync_copy(dstp_hbm.at[pl.ds(b2, _P)], dst_v)
            for b in range(nb - 1):
                gather(b, b).start()

            @pl.loop(0, _P, step=nb)
            def _(j):
                for b in range(nb):
                    cc = j + b
                    tb = (b + nb - 1) % nb
                    gather(cc, b).wait()

                    if b == 0:
                        @pl.when(j > 0)
                        def _():
                            scatter(cc - 1, tb).wait()
                    else:
                        scatter(cc - 1, tb).wait()
                    scatter(cc, b).start()

                    @pl.when(cc + nb - 1 < _P)
                    def _():
                        gather(cc + nb - 1, tb).start()

            scatter(_P - 1, nb - 1).wait()

        @pl.when(c == 0)
        def _():
            for st in range(c0 // _P):
                stage(s * c0 + st * _P)

        @pl.when(c == 1)
        def _():
            for st in range(c1 // _P):
                stage(_NS * c0 + s * c1 + st * _P)

        plsc.subcore_barrier()
        pltpu.sync_copy(acc.at[pl.ds(s * _ROWS_T, _ROWS_T)],
                        out_hbm.at[c, pl.ds(s * _ROWS_T, _ROWS_T)])

    return k(g, srcp, dstp, zeros_blk)


def _dis_body(degp_ref, dis_ref):
    deg = degp_ref[0, :, 0:1] + degp_ref[1, :, 0:1] + 1.0
    dis_ref[...] = lax.rsqrt(deg)


def _dis(degp):
    grid = _NPAD // _ROWS_T
    return pl.pallas_call(
        _dis_body,
        grid=(grid,),
        in_specs=[pl.BlockSpec((_NC, _ROWS_T, 128), lambda i: (0, i, 0))],
        out_specs=pl.BlockSpec((_ROWS_T, 1), lambda i: (i, 0)),
        out_shape=jax.ShapeDtypeStruct((_NPAD, 1), jnp.float32),
    )(degp)


def _mm1_body(x_ref, w_ref, dis_ref, g_ref):
    g_ref[...] = jnp.dot(x_ref[...], w_ref[...],
                         preferred_element_type=jnp.float32) * dis_ref[...]


def _mm1(x, w1, dis):
    grid = _N // _BM
    return pl.pallas_call(
        _mm1_body,
        grid=(grid,),
        in_specs=[
            pl.BlockSpec((_BM, 128), lambda i: (i, 0)),
            pl.BlockSpec((128, 128), lambda i: (0, 0)),
            pl.BlockSpec((_BM, 1), lambda i: (i, 0)),
        ],
        out_specs=pl.BlockSpec((_BM, 128), lambda i: (i, 0)),
        out_shape=jax.ShapeDtypeStruct((_N, 128), jnp.float32),
    )(x, w1, dis)


def _combine_mm_body(parts_ref, g_ref, dis_ref, b_ref, w_ref, o_ref):
    z = dis_ref[...] * (parts_ref[0] + parts_ref[1] + g_ref[...]) + b_ref[...]
    a = jnp.maximum(z, 0.0)
    o_ref[...] = jnp.dot(a, w_ref[...],
                         preferred_element_type=jnp.float32) * dis_ref[...]


def _combine_mm(parts, g, dis, b, w):
    grid = _N // _BM
    fin = g.shape[1]
    fout = w.shape[1]
    return pl.pallas_call(
        _combine_mm_body,
        grid=(grid,),
        in_specs=[
            pl.BlockSpec((_NC, _BM, fin), lambda i: (0, i, 0)),
            pl.BlockSpec((_BM, fin), lambda i: (i, 0)),
            pl.BlockSpec((_BM, 1), lambda i: (i, 0)),
            pl.BlockSpec((1, fin), lambda i: (0, 0)),
            pl.BlockSpec((fin, fout), lambda i: (0, 0)),
        ],
        out_specs=pl.BlockSpec((_BM, fout), lambda i: (i, 0)),
        out_shape=jax.ShapeDtypeStruct((_N, fout), jnp.float32),
    )(parts, g, dis, b, w)


def _combine_scale_body(parts_ref, g_ref, dis_ref, b_ref, o_ref):
    z = dis_ref[...] * (parts_ref[0] + parts_ref[1] + g_ref[...]) + b_ref[...]
    o_ref[...] = jnp.maximum(z, 0.0) * dis_ref[...]


def _combine_scale(parts, g, dis, b):
    grid = _N // _BM
    f = g.shape[1]
    return pl.pallas_call(
        _combine_scale_body,
        grid=(grid,),
        in_specs=[
            pl.BlockSpec((_NC, _BM, f), lambda i: (0, i, 0)),
            pl.BlockSpec((_BM, f), lambda i: (i, 0)),
            pl.BlockSpec((_BM, 1), lambda i: (i, 0)),
            pl.BlockSpec((1, f), lambda i: (0, 0)),
        ],
        out_specs=pl.BlockSpec((_BM, f), lambda i: (i, 0)),
        out_shape=jax.ShapeDtypeStruct((_N, f), jnp.float32),
    )(parts, g, dis, b)


def _final_body(parts_ref, q_ref, dis_ref, w_ref, b_ref, o_ref):
    a = parts_ref[0] + parts_ref[1] + q_ref[...]
    z = jnp.dot(a, w_ref[...], preferred_element_type=jnp.float32)
    z = z * dis_ref[...] + b_ref[...]
    m = jnp.max(z, axis=1, keepdims=True)
    lse = jnp.log(jnp.sum(jnp.exp(z - m), axis=1, keepdims=True)) + m
    o_ref[...] = z - lse


def _final(parts, q, dis, w2, b2):
    grid = _N // _BM
    f = q.shape[1]
    cls = w2.shape[1]
    return pl.pallas_call(
        _final_body,
        grid=(grid,),
        in_specs=[
            pl.BlockSpec((_NC, _BM, f), lambda i: (0, i, 0)),
            pl.BlockSpec((_BM, f), lambda i: (i, 0)),
            pl.BlockSpec((_BM, 1), lambda i: (i, 0)),
            pl.BlockSpec((f, cls), lambda i: (0, 0)),
            pl.BlockSpec((1, cls), lambda i: (0, 0)),
        ],
        out_specs=pl.BlockSpec((_BM, cls), lambda i: (i, 0)),
        out_shape=jax.ShapeDtypeStruct((_N, cls), jnp.float32),
    )(parts, q, dis, w2, b2)


def kernel(x, edge_index, W1, b1, Wg0, bg0, Wg1, bg1, W2, b2):
    src, dst = edge_index[0], edge_index[1]
    npad = _EPAD - _E
    iota = jnp.arange(npad, dtype=jnp.int32)
    srcp = jnp.concatenate([src, iota % _N]).reshape(_CH, _K)
    dstp = jnp.concatenate([dst, _N + iota % (_NPAD - _N)]).reshape(_CH, _K)
    zeros128 = jnp.zeros((_ROWS_T, 128), jnp.float32)
    ones128 = jnp.ones((_K, 128), jnp.float32)

    degp = _deg_parts(dstp, ones128, zeros128)
    dis = _dis(degp)
    g1 = _mm1(x, W1, dis)

    parts = _agg_parts(g1, srcp, dstp, zeros128, 128)
    g2 = _combine_mm(parts, g1, dis, b1.reshape(1, -1), Wg0)

    parts = _agg_parts(g2, srcp, dstp, zeros128, 128)
    g3 = _combine_mm(parts, g2, dis, bg0.reshape(1, -1), Wg1)

    parts = _agg_parts(g3, srcp, dstp, zeros128, 128)
    q = _combine_scale(parts, g3, dis, bg1.reshape(1, -1))

    parts = _agg_parts(q, srcp, dstp, zeros128, 128)
    out = _final(parts, q, dis, W2, b2.reshape(1, -1))
    return out
